# baseline (speedup 1.0000x reference)
import functools

import jax
import jax.numpy as jnp
from jax import lax
from jax.experimental import pallas as pl
from jax.experimental.pallas import tpu as pltpu
from jax.experimental.pallas import tpu_sc as plsc

NC = 2
NS = 16
NW = NC * NS

VOCAB = 1000000
DIM = 64
BATCH = 16384
HIST = 20
BPW = BATCH // NW
SUB = BPW // 2
GRP = 1024
TGRID = (VOCAB + GRP - 1) // GRP
TROWS = TGRID * (GRP // 2)


def _transpose_table(embs_t):
    def body(x_ref, o_ref):
        o_ref[:, 0:DIM] = jnp.transpose(x_ref[:, 0:GRP // 2], (1, 0))
        o_ref[:, DIM:2 * DIM] = jnp.transpose(x_ref[:, GRP // 2:GRP], (1, 0))

    return pl.pallas_call(
        body,
        grid=(TGRID,),
        in_specs=[pl.BlockSpec((DIM, GRP), lambda i: (0, i))],
        out_specs=pl.BlockSpec((GRP // 2, 2 * DIM), lambda i: (i, 0)),
        out_shape=jax.ShapeDtypeStruct((TROWS, 2 * DIM), jnp.float32),
    )(embs_t)


def _make_gather():
    mesh = plsc.VectorSubcoreMesh(core_axis_name="c", subcore_axis_name="s")

    @functools.partial(
        pl.kernel,
        mesh=mesh,
        out_type=jax.ShapeDtypeStruct((HIST * DIM, BATCH), jnp.float32),
        scratch_types=[
            pltpu.VMEM((BPW,), jnp.int32),
            pltpu.VMEM((2, SUB), jnp.int32),
            pltpu.VMEM((SUB, 2 * DIM), jnp.float32),
            pltpu.VMEM((DIM, SUB), jnp.float32),
            pltpu.SemaphoreType.DMA,
        ],
        compiler_params=pltpu.CompilerParams(
            use_tc_tiling_on_sc=False, needs_layout_passes=False),
    )
    def gather(table_hbm, idx_hbm, out_hbm, idx_v, q_v, rows_v, t_v, sem):
        wid = lax.axis_index("s") * NC + lax.axis_index("c")
        b0 = wid * BPW

        def h_body(h, carry):
            pltpu.sync_copy(idx_hbm.at[h, pl.ds(b0, BPW)], idx_v)
            for sub in range(2):
                def q_body(j, c):
                    v = idx_v[pl.ds(sub * SUB + j * 16, 16)]
                    q_v[sub, pl.ds(j * 16, 16)] = (
                        ((v >> 10) << 9) + (v & (GRP // 2 - 1)))
                    return c
                lax.fori_loop(0, SUB // 16, q_body, 0, unroll=4)

                pltpu.async_copy(table_hbm.at[q_v.at[sub]], rows_v, sem).wait()

                def g_body(g, c):
                    iv = idx_v[pl.ds(sub * SUB + g * 16, 16)]
                    hsel = ((iv >> 9) & 1) << 6
                    rowid = lax.iota(jnp.int32, 16) + g * 16
                    def d_body(d, c2):
                        t_v[d, pl.ds(g * 16, 16)] = plsc.load_gather(
                            rows_v, [rowid, hsel + d])
                        return c2
                    return lax.fori_loop(0, DIM, d_body, c, unroll=8)
                lax.fori_loop(0, SUB // 16, g_body, 0)

                pltpu.sync_copy(
                    t_v,
                    out_hbm.at[pl.ds(h * DIM, DIM), pl.ds(b0 + sub * SUB, SUB)])
            return carry

        lax.fori_loop(0, HIST, h_body, 0)

    return gather


def kernel(indices, embs):
    table = _transpose_table(embs.T)
    idx_t = indices.T
    out = _make_gather()(table, idx_t)
    out = out.reshape(HIST, DIM, BATCH)
    return jnp.transpose(out, (2, 0, 1))


# speedup vs baseline: 1.0536x; 1.0536x over previous
import functools

import jax
import jax.numpy as jnp
from jax import lax
from jax.experimental import pallas as pl
from jax.experimental.pallas import tpu as pltpu
from jax.experimental.pallas import tpu_sc as plsc

NC = 2
NS = 16
NW = NC * NS

VOCAB = 1000000
DIM = 64
BATCH = 16384
HIST = 20
BPW = BATCH // NW
SUB = BPW // 2
GRP = 1024
TGRID = (VOCAB + GRP - 1) // GRP
TROWS = TGRID * (GRP // 2)


def _transpose_table(embs_t):
    def body(x_ref, o_ref):
        stacked = jnp.concatenate(
            [x_ref[:, 0:GRP // 2], x_ref[:, GRP // 2:GRP]], axis=0)
        o_ref[...] = jnp.transpose(stacked, (1, 0))

    return pl.pallas_call(
        body,
        grid=(TGRID,),
        in_specs=[pl.BlockSpec((DIM, GRP), lambda i: (0, i))],
        out_specs=pl.BlockSpec((GRP // 2, 2 * DIM), lambda i: (i, 0)),
        out_shape=jax.ShapeDtypeStruct((TROWS, 2 * DIM), jnp.float32),
    )(embs_t)


def _make_gather():
    mesh = plsc.VectorSubcoreMesh(core_axis_name="c", subcore_axis_name="s")

    @functools.partial(
        pl.kernel,
        mesh=mesh,
        out_type=jax.ShapeDtypeStruct((HIST * DIM, BATCH), jnp.float32),
        scratch_types=[
            pltpu.VMEM((BPW,), jnp.int32),
            pltpu.VMEM((2, SUB), jnp.int32),
            pltpu.VMEM((SUB, 2 * DIM), jnp.float32),
            pltpu.VMEM((SUB, 2 * DIM), jnp.float32),
            pltpu.VMEM((DIM, SUB), jnp.float32),
            pltpu.SemaphoreType.DMA,
            pltpu.SemaphoreType.DMA,
        ],
        compiler_params=pltpu.CompilerParams(
            use_tc_tiling_on_sc=False, needs_layout_passes=False),
    )
    def gather(table_hbm, idx_hbm, out_hbm, idx_v, q_v, rows_a, rows_b,
               t_v, sem_a, sem_b):
        wid = lax.axis_index("s") * NC + lax.axis_index("c")
        b0 = wid * BPW
        rows = (rows_a, rows_b)
        sems = (sem_a, sem_b)

        def h_body(h, carry):
            pltpu.sync_copy(idx_hbm.at[h, pl.ds(b0, BPW)], idx_v)
            def q_body(j, c):
                v = idx_v[pl.ds(j * 16, 16)]
                q_v[0, pl.ds(j * 16, 16)] = ((v >> 10) << 9) + (v & (GRP // 2 - 1))
                v = idx_v[pl.ds(SUB + j * 16, 16)]
                q_v[1, pl.ds(j * 16, 16)] = ((v >> 10) << 9) + (v & (GRP // 2 - 1))
                return c
            lax.fori_loop(0, SUB // 16, q_body, 0, unroll=4)

            cps = [pltpu.async_copy(table_hbm.at[q_v.at[s]], rows[s], sems[s])
                   for s in range(2)]
            for sub in range(2):
                cps[sub].wait()
                rv = rows[sub]
                def g_body(g, c):
                    iv = idx_v[pl.ds(sub * SUB + g * 16, 16)]
                    hsel = ((iv >> 9) & 1) << 6
                    rowid = lax.iota(jnp.int32, 16) + g * 16
                    for d in range(DIM):
                        t_v[d, pl.ds(g * 16, 16)] = plsc.load_gather(
                            rv, [rowid, hsel + d])
                    return c
                lax.fori_loop(0, SUB // 16, g_body, 0)

                pltpu.sync_copy(
                    t_v,
                    out_hbm.at[pl.ds(h * DIM, DIM), pl.ds(b0 + sub * SUB, SUB)])
            return carry

        lax.fori_loop(0, HIST, h_body, 0)

    return gather


def kernel(indices, embs):
    table = _transpose_table(embs.T)
    idx_t = indices.T
    out = _make_gather()(table, idx_t)
    out = out.reshape(HIST, DIM, BATCH)
    return jnp.transpose(out, (2, 0, 1))


# speedup vs baseline: 2.0404x; 1.9367x over previous
import functools

import jax
import jax.numpy as jnp
from jax import lax
from jax.experimental import pallas as pl
from jax.experimental.pallas import tpu as pltpu
from jax.experimental.pallas import tpu_sc as plsc

NC = 2
NS = 16
NW = NC * NS

VOCAB = 1000000
DIM = 64
BATCH = 16384
HIST = 20
BPW = BATCH // NW
SUB = BPW // 2
GRP = 1024
TGRID = (VOCAB + GRP - 1) // GRP
TROWS = TGRID * (GRP // 2)


def _transpose_table(embs_t):
    gpb = 8

    def body(x_ref, o_ref):
        for g in range(gpb):
            stacked = jnp.concatenate(
                [x_ref[:, g * GRP:g * GRP + GRP // 2],
                 x_ref[:, g * GRP + GRP // 2:(g + 1) * GRP]], axis=0)
            o_ref[g * (GRP // 2):(g + 1) * (GRP // 2), :] = (
                jnp.transpose(stacked, (1, 0)))

    grid = (TGRID + gpb - 1) // gpb
    return pl.pallas_call(
        body,
        grid=(grid,),
        in_specs=[pl.BlockSpec((DIM, gpb * GRP), lambda i: (0, i))],
        out_specs=pl.BlockSpec((gpb * GRP // 2, 2 * DIM), lambda i: (i, 0)),
        out_shape=jax.ShapeDtypeStruct((TROWS, 2 * DIM), jnp.float32),
    )(embs_t)


def _make_gather():
    mesh = plsc.VectorSubcoreMesh(core_axis_name="c", subcore_axis_name="s")

    @functools.partial(
        pl.kernel,
        mesh=mesh,
        out_type=jax.ShapeDtypeStruct((HIST * DIM, BATCH), jnp.float32),
        scratch_types=[
            pltpu.VMEM((BPW,), jnp.int32),
            pltpu.VMEM((2, SUB), jnp.int32),
            pltpu.VMEM((SUB, 2 * DIM), jnp.float32),
            pltpu.VMEM((SUB, 2 * DIM), jnp.float32),
            pltpu.VMEM((DIM, SUB), jnp.float32),
            pltpu.SemaphoreType.DMA,
            pltpu.SemaphoreType.DMA,
        ],
        compiler_params=pltpu.CompilerParams(
            use_tc_tiling_on_sc=False, needs_layout_passes=False),
    )
    def gather(table_hbm, idx_hbm, out_hbm, idx_v, q_v, rows_a, rows_b,
               t_v, sem_a, sem_b):
        wid = lax.axis_index("s") * NC + lax.axis_index("c")
        b0 = wid * BPW
        rows = (rows_a, rows_b)
        sems = (sem_a, sem_b)

        def h_body(h, carry):
            pltpu.sync_copy(idx_hbm.at[h, pl.ds(b0, BPW)], idx_v)
            def q_body(j, c):
                v = idx_v[pl.ds(j * 16, 16)]
                q_v[0, pl.ds(j * 16, 16)] = ((v >> 10) << 9) + (v & (GRP // 2 - 1))
                v = idx_v[pl.ds(SUB + j * 16, 16)]
                q_v[1, pl.ds(j * 16, 16)] = ((v >> 10) << 9) + (v & (GRP // 2 - 1))
                return c
            lax.fori_loop(0, SUB // 16, q_body, 0, unroll=4)

            cps = [pltpu.async_copy(table_hbm.at[q_v.at[s]], rows[s], sems[s])
                   for s in range(2)]
            for sub in range(2):
                cps[sub].wait()
                rv = rows[sub]
                @plsc.parallel_loop(0, SUB // 16)
                def g_body(g, _sub=sub, _rv=rv):
                    iv = idx_v[pl.ds(_sub * SUB + g * 16, 16)]
                    hsel = ((iv >> 9) & 1) << 6
                    rowid = lax.iota(jnp.int32, 16) + g * 16
                    for d in range(DIM):
                        t_v[d, pl.ds(g * 16, 16)] = plsc.load_gather(
                            _rv, [rowid, hsel + d])

                pltpu.sync_copy(
                    t_v,
                    out_hbm.at[pl.ds(h * DIM, DIM), pl.ds(b0 + sub * SUB, SUB)])
            return carry

        lax.fori_loop(0, HIST, h_body, 0)

    return gather


def kernel(indices, embs):
    table = _transpose_table(embs.T)
    idx_t = indices.T
    out = _make_gather()(table, idx_t)
    out = out.reshape(HIST, DIM, BATCH)
    return jnp.transpose(out, (2, 0, 1))


# speedup vs baseline: 2.2471x; 1.1013x over previous
import functools

import jax
import jax.numpy as jnp
from jax import lax
from jax.experimental import pallas as pl
from jax.experimental.pallas import tpu as pltpu
from jax.experimental.pallas import tpu_sc as plsc

NC = 2
NS = 16
NW = NC * NS

VOCAB = 1000000
DIM = 64
BATCH = 16384
HIST = 20
BPW = BATCH // NW
SUB = BPW // 2
GRP = 1024
TGRID = (VOCAB + GRP - 1) // GRP
TROWS = TGRID * (GRP // 2)


def _transpose_table(embs_t):
    gpb = 8

    def body(x_ref, o_ref):
        for g in range(gpb):
            stacked = jnp.concatenate(
                [x_ref[:, g * GRP:g * GRP + GRP // 2],
                 x_ref[:, g * GRP + GRP // 2:(g + 1) * GRP]], axis=0)
            o_ref[g * (GRP // 2):(g + 1) * (GRP // 2), :] = (
                jnp.transpose(stacked, (1, 0)))

    grid = (TGRID + gpb - 1) // gpb
    return pl.pallas_call(
        body,
        grid=(grid,),
        in_specs=[pl.BlockSpec((DIM, gpb * GRP), lambda i: (0, i))],
        out_specs=pl.BlockSpec((gpb * GRP // 2, 2 * DIM), lambda i: (i, 0)),
        out_shape=jax.ShapeDtypeStruct((TROWS, 2 * DIM), jnp.float32),
    )(embs_t)


def _make_gather():
    mesh = plsc.VectorSubcoreMesh(core_axis_name="c", subcore_axis_name="s")

    @functools.partial(
        pl.kernel,
        mesh=mesh,
        out_type=jax.ShapeDtypeStruct((HIST * DIM, BATCH), jnp.float32),
        scratch_types=[
            pltpu.VMEM((BPW,), jnp.int32),
            pltpu.VMEM((2, SUB), jnp.int32),
            pltpu.VMEM((SUB, DIM), jnp.float32),
            pltpu.VMEM((SUB, DIM), jnp.float32),
            pltpu.VMEM((DIM, SUB), jnp.float32),
            pltpu.SemaphoreType.DMA,
            pltpu.SemaphoreType.DMA,
        ],
        compiler_params=pltpu.CompilerParams(
            use_tc_tiling_on_sc=False, needs_layout_passes=False),
    )
    def gather(table_hbm, idx_hbm, out_hbm, idx_v, q_v, rows_a, rows_b,
               t_v, sem_a, sem_b):
        wid = lax.axis_index("s") * NC + lax.axis_index("c")
        b0 = wid * BPW
        rows = (rows_a, rows_b)
        sems = (sem_a, sem_b)

        def load_q(h, sub):
            def q_body(j, c):
                v = idx_v[pl.ds(sub * SUB + j * 16, 16)]
                q_v[sub, pl.ds(j * 16, 16)] = (
                    ((v >> 10) << 10) + ((v & (GRP // 2 - 1)) << 1)
                    + ((v >> 9) & 1))
                return c
            lax.fori_loop(0, SUB // 16, q_body, 0, unroll=4)

        def fire(sub):
            return pltpu.async_copy(table_hbm.at[q_v.at[sub]], rows[sub],
                                    sems[sub])

        def drain(sub):
            pltpu.make_async_copy(table_hbm.at[q_v.at[sub]], rows[sub],
                                  sems[sub]).wait()

        def transpose_out(h, sub):
            rv = rows[sub]
            @plsc.parallel_loop(0, SUB // 16)
            def g_body(g, _rv=rv):
                rowid = lax.iota(jnp.int32, 16) + g * 16
                for d in range(DIM):
                    t_v[d, pl.ds(g * 16, 16)] = plsc.load_gather(
                        _rv, [rowid, jnp.full((16,), d, jnp.int32)])

            pltpu.sync_copy(
                t_v, out_hbm.at[pl.ds(h * DIM, DIM), pl.ds(b0 + sub * SUB, SUB)])

        pltpu.sync_copy(idx_hbm.at[0, pl.ds(b0, BPW)], idx_v)
        load_q(0, 0)
        load_q(0, 1)
        fire(0)

        def h_body(h, carry):
            fire(1)
            drain(0)
            transpose_out(h, 0)
            @pl.when(h + 1 < HIST)
            def _():
                pltpu.sync_copy(idx_hbm.at[h + 1, pl.ds(b0, BPW)], idx_v)
                load_q(h + 1, 0)
                fire(0)
            drain(1)
            transpose_out(h, 1)
            @pl.when(h + 1 < HIST)
            def _():
                load_q(h + 1, 1)
            return carry

        lax.fori_loop(0, HIST, h_body, 0)

    return gather


def kernel(indices, embs):
    table = _transpose_table(embs.T)
    table = table.reshape(2 * TROWS, DIM)
    idx_t = indices.T
    out = _make_gather()(table, idx_t)
    out = out.reshape(HIST, DIM, BATCH)
    return jnp.transpose(out, (2, 0, 1))


# speedup vs baseline: 3.7819x; 1.6830x over previous
import functools

import jax
import jax.numpy as jnp
from jax import lax
from jax.experimental import pallas as pl
from jax.experimental.pallas import tpu as pltpu
from jax.experimental.pallas import tpu_sc as plsc

NC = 2
NS = 16
NW = NC * NS

VOCAB = 1000000
DIM = 64
BATCH = 16384
HIST = 20
BPW = BATCH // NW
SUB = BPW // 2
GRP = 1024
TGRID = (VOCAB + GRP - 1) // GRP
TROWS = TGRID * (GRP // 2)


def _transpose_table(embs_t):
    gpb = 8

    def body(x_ref, o_ref):
        for g in range(gpb):
            stacked = jnp.concatenate(
                [x_ref[:, g * GRP:g * GRP + GRP // 2],
                 x_ref[:, g * GRP + GRP // 2:(g + 1) * GRP]], axis=0)
            o_ref[g * (GRP // 2):(g + 1) * (GRP // 2), :] = (
                jnp.transpose(stacked, (1, 0)))

    grid = (TGRID + gpb - 1) // gpb
    return pl.pallas_call(
        body,
        grid=(grid,),
        in_specs=[pl.BlockSpec((DIM, gpb * GRP), lambda i: (0, i))],
        out_specs=pl.BlockSpec((gpb * GRP // 2, 2 * DIM), lambda i: (i, 0)),
        out_shape=jax.ShapeDtypeStruct((TROWS, 2 * DIM), jnp.float32),
    )(embs_t)


def _make_gather():
    mesh = plsc.VectorSubcoreMesh(core_axis_name="c", subcore_axis_name="s")

    @functools.partial(
        pl.kernel,
        mesh=mesh,
        out_type=jax.ShapeDtypeStruct((HIST * DIM, BATCH), jnp.float32),
        scratch_types=[
            pltpu.VMEM((BPW,), jnp.int32),
            pltpu.VMEM((2, SUB), jnp.int32),
            pltpu.VMEM((SUB, DIM), jnp.float32),
            pltpu.VMEM((SUB, DIM), jnp.float32),
            pltpu.VMEM((DIM, SUB + 1), jnp.float32),
            pltpu.SemaphoreType.DMA,
            pltpu.SemaphoreType.DMA,
        ],
        compiler_params=pltpu.CompilerParams(
            use_tc_tiling_on_sc=False, needs_layout_passes=False),
    )
    def gather(table_hbm, idx_hbm, out_hbm, idx_v, q_v, rows_a, rows_b,
               t_v, sem_a, sem_b):
        wid = lax.axis_index("s") * NC + lax.axis_index("c")
        b0 = wid * BPW
        rows = (rows_a, rows_b)
        sems = (sem_a, sem_b)

        def load_q(h, sub):
            def q_body(j, c):
                v = idx_v[pl.ds(sub * SUB + j * 16, 16)]
                q_v[sub, pl.ds(j * 16, 16)] = (
                    ((v >> 10) << 10) + ((v & (GRP // 2 - 1)) << 1)
                    + ((v >> 9) & 1))
                return c
            lax.fori_loop(0, SUB // 16, q_body, 0, unroll=4)

        def fire(sub):
            return pltpu.async_copy(table_hbm.at[q_v.at[sub]], rows[sub],
                                    sems[sub])

        def drain(sub):
            pltpu.make_async_copy(table_hbm.at[q_v.at[sub]], rows[sub],
                                  sems[sub]).wait()

        dvecs = [lax.iota(jnp.int32, 16) + 16 * k for k in range(DIM // 16)]

        def transpose_out(h, sub):
            rv = rows[sub]
            @plsc.parallel_loop(0, SUB, unroll=2)
            def b_body(b, _rv=rv):
                bvec = jnp.full((16,), 0, jnp.int32) + b
                for k in range(DIM // 16):
                    plsc.store_scatter(t_v, [dvecs[k], bvec],
                                       _rv[b, pl.ds(16 * k, 16)])

            pltpu.sync_copy(
                t_v.at[:, pl.ds(0, SUB)],
                out_hbm.at[pl.ds(h * DIM, DIM), pl.ds(b0 + sub * SUB, SUB)])

        pltpu.sync_copy(idx_hbm.at[0, pl.ds(b0, BPW)], idx_v)
        load_q(0, 0)
        load_q(0, 1)
        fire(0)

        def h_body(h, carry):
            fire(1)
            drain(0)
            transpose_out(h, 0)
            @pl.when(h + 1 < HIST)
            def _():
                pltpu.sync_copy(idx_hbm.at[h + 1, pl.ds(b0, BPW)], idx_v)
                load_q(h + 1, 0)
                fire(0)
            drain(1)
            transpose_out(h, 1)
            @pl.when(h + 1 < HIST)
            def _():
                load_q(h + 1, 1)
            return carry

        lax.fori_loop(0, HIST, h_body, 0)

    return gather


def kernel(indices, embs):
    table = _transpose_table(embs.T)
    table = table.reshape(2 * TROWS, DIM)
    idx_t = indices.T
    out = _make_gather()(table, idx_t)
    out = out.reshape(HIST, DIM, BATCH)
    return jnp.transpose(out, (2, 0, 1))


# speedup vs baseline: 4.0843x; 1.0800x over previous
import functools

import jax
import jax.numpy as jnp
from jax import lax
from jax.experimental import pallas as pl
from jax.experimental.pallas import tpu as pltpu
from jax.experimental.pallas import tpu_sc as plsc

NC = 2
NS = 16
NW = NC * NS

VOCAB = 1000000
DIM = 64
BATCH = 16384
HIST = 20
BPW = BATCH // NW
SUB = BPW // 2
GRP = 1024
TGRID = (VOCAB + GRP - 1) // GRP
TROWS = TGRID * (GRP // 2)


def _transpose_table(embs_t):
    gpb = 16

    def body(x_ref, o_ref):
        for g in range(gpb):
            stacked = jnp.concatenate(
                [x_ref[:, g * GRP:g * GRP + GRP // 2],
                 x_ref[:, g * GRP + GRP // 2:(g + 1) * GRP]], axis=0)
            o_ref[g * (GRP // 2):(g + 1) * (GRP // 2), :] = (
                jnp.transpose(stacked, (1, 0)))

    grid = (TGRID + gpb - 1) // gpb
    return pl.pallas_call(
        body,
        grid=(grid,),
        in_specs=[pl.BlockSpec((DIM, gpb * GRP), lambda i: (0, i))],
        out_specs=pl.BlockSpec((gpb * GRP // 2, 2 * DIM), lambda i: (i, 0)),
        out_shape=jax.ShapeDtypeStruct((TROWS, 2 * DIM), jnp.float32),
    )(embs_t)


def _make_gather():
    mesh = plsc.VectorSubcoreMesh(core_axis_name="c", subcore_axis_name="s")

    @functools.partial(
        pl.kernel,
        mesh=mesh,
        out_type=jax.ShapeDtypeStruct((HIST * DIM, BATCH), jnp.float32),
        scratch_types=[
            pltpu.VMEM((BPW,), jnp.int32),
            pltpu.VMEM((2, SUB), jnp.int32),
            pltpu.VMEM((SUB, DIM), jnp.float32),
            pltpu.VMEM((SUB, DIM), jnp.float32),
            pltpu.VMEM((DIM, SUB + 1), jnp.float32),
            pltpu.SemaphoreType.DMA,
            pltpu.SemaphoreType.DMA,
        ],
        compiler_params=pltpu.CompilerParams(
            use_tc_tiling_on_sc=False, needs_layout_passes=False),
    )
    def gather(table_hbm, idx_hbm, out_hbm, idx_v, q_v, rows_a, rows_b,
               t_v, sem_a, sem_b):
        wid = lax.axis_index("s") * NC + lax.axis_index("c")
        b0 = wid * BPW
        rows = (rows_a, rows_b)
        sems = (sem_a, sem_b)

        def load_q(h, sub):
            def q_body(j, c):
                v = idx_v[pl.ds(sub * SUB + j * 16, 16)]
                q_v[sub, pl.ds(j * 16, 16)] = (
                    ((v >> 10) << 10) + ((v & (GRP // 2 - 1)) << 1)
                    + ((v >> 9) & 1))
                return c
            lax.fori_loop(0, SUB // 16, q_body, 0, unroll=4)

        def fire(sub):
            return pltpu.async_copy(table_hbm.at[q_v.at[sub]], rows[sub],
                                    sems[sub])

        def drain(sub):
            pltpu.make_async_copy(table_hbm.at[q_v.at[sub]], rows[sub],
                                  sems[sub]).wait()

        dvecs = [lax.iota(jnp.int32, 16) + 16 * k for k in range(DIM // 16)]

        def transpose_out(h, sub):
            rv = rows[sub]
            @plsc.parallel_loop(0, SUB, unroll=2)
            def b_body(b, _rv=rv):
                bvec = jnp.full((16,), 0, jnp.int32) + b
                for k in range(DIM // 16):
                    plsc.store_scatter(t_v, [dvecs[k], bvec],
                                       _rv[b, pl.ds(16 * k, 16)])

            pltpu.sync_copy(
                t_v.at[:, pl.ds(0, SUB)],
                out_hbm.at[pl.ds(h * DIM, DIM), pl.ds(b0 + sub * SUB, SUB)])

        pltpu.sync_copy(idx_hbm.at[0, pl.ds(b0, BPW)], idx_v)
        load_q(0, 0)
        load_q(0, 1)
        fire(0)

        def h_body(h, carry):
            fire(1)
            drain(0)
            transpose_out(h, 0)
            @pl.when(h + 1 < HIST)
            def _():
                pltpu.sync_copy(idx_hbm.at[h + 1, pl.ds(b0, BPW)], idx_v)
                load_q(h + 1, 0)
                fire(0)
            drain(1)
            transpose_out(h, 1)
            @pl.when(h + 1 < HIST)
            def _():
                load_q(h + 1, 1)
            return carry

        lax.fori_loop(0, HIST, h_body, 0)

    return gather


def kernel(indices, embs):
    table = _transpose_table(embs.T)
    table = table.reshape(2 * TROWS, DIM)
    idx_t = indices.T
    out = _make_gather()(table, idx_t)
    out = out.reshape(HIST, DIM, BATCH)
    return jnp.transpose(out, (2, 0, 1))


# speedup vs baseline: 4.1624x; 1.0191x over previous
import functools

import jax
import jax.numpy as jnp
from jax import lax
from jax.experimental import pallas as pl
from jax.experimental.pallas import tpu as pltpu
from jax.experimental.pallas import tpu_sc as plsc

NC = 2
NS = 16
NW = NC * NS

VOCAB = 1000000
DIM = 64
BATCH = 16384
HIST = 20
BPW = BATCH // NW
SUB = BPW // 2
GRP = 1024
TGRID = (VOCAB + GRP - 1) // GRP
TROWS = TGRID * (GRP // 2)


def _transpose_table(embs_t):
    gpb = 32

    def body(x_ref, o_ref):
        for g in range(gpb):
            stacked = jnp.concatenate(
                [x_ref[:, g * GRP:g * GRP + GRP // 2],
                 x_ref[:, g * GRP + GRP // 2:(g + 1) * GRP]], axis=0)
            o_ref[g * (GRP // 2):(g + 1) * (GRP // 2), :] = (
                jnp.transpose(stacked, (1, 0)))

    grid = (TGRID + gpb - 1) // gpb
    return pl.pallas_call(
        body,
        grid=(grid,),
        in_specs=[pl.BlockSpec((DIM, gpb * GRP), lambda i: (0, i))],
        out_specs=pl.BlockSpec((gpb * GRP // 2, 2 * DIM), lambda i: (i, 0)),
        out_shape=jax.ShapeDtypeStruct((TROWS, 2 * DIM), jnp.float32),
    )(embs_t)


def _make_gather():
    mesh = plsc.VectorSubcoreMesh(core_axis_name="c", subcore_axis_name="s")

    @functools.partial(
        pl.kernel,
        mesh=mesh,
        out_type=jax.ShapeDtypeStruct((HIST * DIM, BATCH), jnp.float32),
        scratch_types=[
            pltpu.VMEM((BPW,), jnp.int32),
            pltpu.VMEM((2, SUB), jnp.int32),
            pltpu.VMEM((SUB, DIM), jnp.float32),
            pltpu.VMEM((SUB, DIM), jnp.float32),
            pltpu.VMEM((DIM, SUB + 1), jnp.float32),
            pltpu.SemaphoreType.DMA,
            pltpu.SemaphoreType.DMA,
        ],
        compiler_params=pltpu.CompilerParams(
            use_tc_tiling_on_sc=False, needs_layout_passes=False),
    )
    def gather(table_hbm, idx_hbm, out_hbm, idx_v, q_v, rows_a, rows_b,
               t_v, sem_a, sem_b):
        wid = lax.axis_index("s") * NC + lax.axis_index("c")
        b0 = wid * BPW
        rows = (rows_a, rows_b)
        sems = (sem_a, sem_b)

        def load_q(h, sub):
            def q_body(j, c):
                v = idx_v[pl.ds(sub * SUB + j * 16, 16)]
                q_v[sub, pl.ds(j * 16, 16)] = (
                    ((v >> 10) << 10) + ((v & (GRP // 2 - 1)) << 1)
                    + ((v >> 9) & 1))
                return c
            lax.fori_loop(0, SUB // 16, q_body, 0, unroll=4)

        def fire(sub):
            return pltpu.async_copy(table_hbm.at[q_v.at[sub]], rows[sub],
                                    sems[sub])

        def drain(sub):
            pltpu.make_async_copy(table_hbm.at[q_v.at[sub]], rows[sub],
                                  sems[sub]).wait()

        dvecs = [lax.iota(jnp.int32, 16) + 16 * k for k in range(DIM // 16)]

        def transpose_out(h, sub):
            rv = rows[sub]
            @plsc.parallel_loop(0, SUB, unroll=2)
            def b_body(b, _rv=rv):
                bvec = jnp.full((16,), 0, jnp.int32) + b
                for k in range(DIM // 16):
                    plsc.store_scatter(t_v, [dvecs[k], bvec],
                                       _rv[b, pl.ds(16 * k, 16)])

            pltpu.sync_copy(
                t_v.at[:, pl.ds(0, SUB)],
                out_hbm.at[pl.ds(h * DIM, DIM), pl.ds(b0 + sub * SUB, SUB)])

        pltpu.sync_copy(idx_hbm.at[0, pl.ds(b0, BPW)], idx_v)
        load_q(0, 0)
        load_q(0, 1)
        fire(0)

        def h_body(h, carry):
            fire(1)
            drain(0)
            transpose_out(h, 0)
            @pl.when(h + 1 < HIST)
            def _():
                pltpu.sync_copy(idx_hbm.at[h + 1, pl.ds(b0, BPW)], idx_v)
                load_q(h + 1, 0)
                fire(0)
            drain(1)
            transpose_out(h, 1)
            @pl.when(h + 1 < HIST)
            def _():
                load_q(h + 1, 1)
            return carry

        lax.fori_loop(0, HIST, h_body, 0)

    return gather


def kernel(indices, embs):
    table = _transpose_table(embs.T)
    table = table.reshape(2 * TROWS, DIM)
    idx_t = indices.T
    out = _make_gather()(table, idx_t)
    out = out.reshape(HIST, DIM, BATCH)
    return jnp.transpose(out, (2, 0, 1))
